# baseline (speedup 1.0000x reference)
import jax
import jax.numpy as jnp
from jax.experimental import pallas as pl
from jax.experimental.pallas import tpu as pltpu

B = 8192
HID = 2048
P = 1024
K = 8
TOPK = 2

BM = 256


def _moe_body(h_ref, wg_ref, w_ref, b_ref, out_ref):
    h32 = h_ref[...]
    gate = jax.lax.dot(h32, wg_ref[...], preferred_element_type=jnp.float32)

    iota = jax.lax.broadcasted_iota(jnp.int32, gate.shape, 1)
    v1 = jnp.max(gate, axis=1, keepdims=True)
    i1 = jnp.min(jnp.where(gate == v1, iota, K), axis=1, keepdims=True)
    masked = jnp.where(iota == i1, -jnp.inf, gate)
    v2 = jnp.max(masked, axis=1, keepdims=True)
    i2 = jnp.min(jnp.where(masked == v2, iota, K), axis=1, keepdims=True)
    t = jnp.exp(v2 - v1)
    w1 = 1.0 / (1.0 + t)
    w2 = t / (1.0 + t)
    weights = (jnp.where(iota == i1, w1, 0.0)
               + jnp.where(iota == i2, w2, 0.0))

    hb = h32.astype(jnp.bfloat16)
    acc = jax.lax.dot(weights, b_ref[...], preferred_element_type=jnp.float32)
    for k in range(K):
        yk = jax.lax.dot(
            hb, w_ref[:, k * P:(k + 1) * P], preferred_element_type=jnp.float32
        )
        acc = acc + weights[:, k:k + 1] * yk
    out_ref[...] = acc


@jax.jit
def kernel(h, W_exp, b_exp, W_gate):
    Wb = W_exp.astype(jnp.bfloat16)
    b2 = b_exp.reshape(K, P)
    grid = (B // BM,)
    return pl.pallas_call(
        _moe_body,
        grid=grid,
        in_specs=[
            pl.BlockSpec((BM, HID), lambda i: (i, 0)),
            pl.BlockSpec((HID, K), lambda i: (0, 0)),
            pl.BlockSpec((HID, K * P), lambda i: (0, 0)),
            pl.BlockSpec((K, P), lambda i: (0, 0)),
        ],
        out_specs=pl.BlockSpec((BM, P), lambda i: (i, 0)),
        out_shape=jax.ShapeDtypeStruct((B, P), jnp.float32),
        compiler_params=pltpu.CompilerParams(
            vmem_limit_bytes=61 * 1024 * 1024,
        ),
    )(h, W_gate, Wb, b2)


# speedup vs baseline: 1.0164x; 1.0164x over previous
import jax
import jax.numpy as jnp
from jax.experimental import pallas as pl
from jax.experimental.pallas import tpu as pltpu

B = 8192
HID = 2048
P = 1024
K = 8
TOPK = 2

BM = 512


def _moe_body(h_ref, wg_ref, w_ref, b_ref, out_ref):
    h32 = h_ref[...]
    gate = jax.lax.dot(h32, wg_ref[...], preferred_element_type=jnp.float32)

    iota = jax.lax.broadcasted_iota(jnp.int32, gate.shape, 1)
    v1 = jnp.max(gate, axis=1, keepdims=True)
    i1 = jnp.min(jnp.where(gate == v1, iota, K), axis=1, keepdims=True)
    masked = jnp.where(iota == i1, -jnp.inf, gate)
    v2 = jnp.max(masked, axis=1, keepdims=True)
    i2 = jnp.min(jnp.where(masked == v2, iota, K), axis=1, keepdims=True)
    t = jnp.exp(v2 - v1)
    w1 = 1.0 / (1.0 + t)
    w2 = t / (1.0 + t)
    weights = (jnp.where(iota == i1, w1, 0.0)
               + jnp.where(iota == i2, w2, 0.0))

    hb = h32.astype(jnp.bfloat16)
    acc = jax.lax.dot(weights, b_ref[...], preferred_element_type=jnp.float32)
    for k in range(K):
        yk = jax.lax.dot(
            hb, w_ref[:, k * P:(k + 1) * P], preferred_element_type=jnp.float32
        )
        acc = acc + weights[:, k:k + 1] * yk
    out_ref[...] = acc


@jax.jit
def kernel(h, W_exp, b_exp, W_gate):
    Wb = W_exp.astype(jnp.bfloat16)
    b2 = b_exp.reshape(K, P)
    grid = (B // BM,)
    return pl.pallas_call(
        _moe_body,
        grid=grid,
        in_specs=[
            pl.BlockSpec((BM, HID), lambda i: (i, 0)),
            pl.BlockSpec((HID, K), lambda i: (0, 0)),
            pl.BlockSpec((HID, K * P), lambda i: (0, 0)),
            pl.BlockSpec((K, P), lambda i: (0, 0)),
        ],
        out_specs=pl.BlockSpec((BM, P), lambda i: (i, 0)),
        out_shape=jax.ShapeDtypeStruct((B, P), jnp.float32),
        compiler_params=pltpu.CompilerParams(
            vmem_limit_bytes=61 * 1024 * 1024,
        ),
    )(h, W_gate, Wb, b2)


# speedup vs baseline: 1.0211x; 1.0047x over previous
import jax
import jax.numpy as jnp
from jax.experimental import pallas as pl
from jax.experimental.pallas import tpu as pltpu

B = 8192
HID = 2048
P = 1024
K = 8
TOPK = 2

BM = 512
KH = 4
CW = 512
CPE = P // CW
NCHUNK = (K - KH) * CPE


def _moe_body(h_ref, wg_ref, whi_ref, w_hbm, b_ref, out_ref, w2_vmem, stage, sems):
    first = pl.program_id(0) == 0

    def _copy(c, buf):
        col = (KH + c // CPE) * P + (c % CPE) * CW
        return pltpu.make_async_copy(
            w_hbm.at[:, pl.ds(col, CW)], stage.at[buf], sems.at[buf]
        )

    @pl.when(first)
    def _prime():
        _copy(0, 0).start()
        _copy(1, 1).start()

    h32 = h_ref[...]
    gate = jax.lax.dot(h32, wg_ref[...], preferred_element_type=jnp.float32)

    iota = jax.lax.broadcasted_iota(jnp.int32, gate.shape, 1)
    v1 = jnp.max(gate, axis=1, keepdims=True)
    i1 = jnp.min(jnp.where(gate == v1, iota, K), axis=1, keepdims=True)
    masked = jnp.where(iota == i1, -jnp.inf, gate)
    v2 = jnp.max(masked, axis=1, keepdims=True)
    i2 = jnp.min(jnp.where(masked == v2, iota, K), axis=1, keepdims=True)
    t = jnp.exp(v2 - v1)
    w1 = 1.0 / (1.0 + t)
    w2 = t / (1.0 + t)
    weights = (jnp.where(iota == i1, w1, 0.0)
               + jnp.where(iota == i2, w2, 0.0))

    hb = h32.astype(jnp.bfloat16)
    acc = jax.lax.dot(weights, b_ref[...], preferred_element_type=jnp.float32)
    for k in range(K):
        if k < KH:
            wk_mat = whi_ref[:, k * P:(k + 1) * P]
        else:
            @pl.when(first)
            def _convert(k=k):
                for c in range((k - KH) * CPE, (k - KH + 1) * CPE):
                    _copy(c, c % 2).wait()
                    w2_vmem[k - KH, :, pl.ds((c % CPE) * CW, CW)] = (
                        stage[c % 2].astype(jnp.bfloat16))
                    if c + 2 < NCHUNK:
                        _copy(c + 2, c % 2).start()

            wk_mat = w2_vmem[k - KH]
        yk = jax.lax.dot(hb, wk_mat, preferred_element_type=jnp.float32)
        acc = acc + weights[:, k:k + 1] * yk
    out_ref[...] = acc


@jax.jit
def kernel(h, W_exp, b_exp, W_gate):
    Whi = W_exp[:, :KH * P].astype(jnp.bfloat16)
    b2 = b_exp.reshape(K, P)
    grid = (B // BM,)
    return pl.pallas_call(
        _moe_body,
        grid=grid,
        in_specs=[
            pl.BlockSpec((BM, HID), lambda i: (i, 0)),
            pl.BlockSpec((HID, K), lambda i: (0, 0)),
            pl.BlockSpec((HID, KH * P), lambda i: (0, 0)),
            pl.BlockSpec(memory_space=pltpu.MemorySpace.HBM),
            pl.BlockSpec((K, P), lambda i: (0, 0)),
        ],
        out_specs=pl.BlockSpec((BM, P), lambda i: (i, 0)),
        out_shape=jax.ShapeDtypeStruct((B, P), jnp.float32),
        scratch_shapes=[
            pltpu.VMEM((K - KH, HID, P), jnp.bfloat16),
            pltpu.VMEM((2, HID, CW), jnp.float32),
            pltpu.SemaphoreType.DMA((2,)),
        ],
        compiler_params=pltpu.CompilerParams(
            vmem_limit_bytes=61 * 1024 * 1024,
        ),
    )(h, W_gate, Whi, W_exp, b2)
